# Initial kernel scaffold; baseline (speedup 1.0000x reference)
import jax
import jax.numpy as jnp
from jax.experimental import pallas as pl


def kernel(boxes, classification):
    raise NotImplementedError("write your pallas kernel here")


# speedup vs baseline: 13.8272x; 13.8272x over previous
import jax
import jax.numpy as jnp
from jax import lax
from jax.experimental import pallas as pl
from jax.experimental.pallas import tpu as pltpu

NMS_THRESHOLD = 0.5
SCORE_THRESHOLD = 0.05
MAXDET = 300
NEG = -1e9
B, N, C = 2, 5000, 20
NP = 5120
PD = 304
R = B * C


def _fd_kernel(s_in, x1_in, y1_in, x2_in, y2_in,
               boxes_out, scoresT_out, labelsT_out,
               s_ref, area_ref, sel_s,
               sel_x1, sel_y1, sel_x2, sel_y2):
    f32 = jnp.float32
    i32 = jnp.int32
    iota_n = lax.broadcasted_iota(i32, (R, NP), 1)
    lane_pd = lax.broadcasted_iota(i32, (R, PD), 1)

    x1 = x1_in[...]
    y1 = y1_in[...]
    x2 = x2_in[...]
    y2 = y2_in[...]
    raw = s_in[...]
    s_ref[...] = jnp.where(raw > SCORE_THRESHOLD, raw, NEG)
    area_ref[...] = (x2 - x1) * (y2 - y1)
    sel_s[...] = jnp.full((R, PD), NEG, f32)
    sel_x1[...] = jnp.zeros((R, PD), f32)
    sel_y1[...] = jnp.zeros((R, PD), f32)
    sel_x2[...] = jnp.zeros((R, PD), f32)
    sel_y2[...] = jnp.zeros((R, PD), f32)
    area = area_ref[...]

    def nms_step(t, carry):
        s = s_ref[...]
        m = jnp.max(s, axis=1, keepdims=True)
        eq = s == m
        idx = jnp.min(jnp.where(eq, iota_n, NP), axis=1, keepdims=True)
        onehot = iota_n == idx
        bx1 = jnp.sum(jnp.where(onehot, x1, 0.0), axis=1, keepdims=True)
        by1 = jnp.sum(jnp.where(onehot, y1, 0.0), axis=1, keepdims=True)
        bx2 = jnp.sum(jnp.where(onehot, x2, 0.0), axis=1, keepdims=True)
        by2 = jnp.sum(jnp.where(onehot, y2, 0.0), axis=1, keepdims=True)
        valid = m > (NEG * 0.5)
        xx1 = jnp.maximum(bx1, x1)
        yy1 = jnp.maximum(by1, y1)
        xx2 = jnp.minimum(bx2, x2)
        yy2 = jnp.minimum(by2, y2)
        inter = jnp.maximum(xx2 - xx1, 0.0) * jnp.maximum(yy2 - yy1, 0.0)
        area_a = (bx2 - bx1) * (by2 - by1)
        union = area_a + area - inter
        iou = jnp.where(union > 0.0, inter / union, 0.0)
        supp = iou > NMS_THRESHOLD
        s_new = jnp.where(valid & supp, NEG, s)
        s_new = jnp.where(onehot, NEG, s_new)
        s_ref[...] = s_new
        tm = lane_pd == t
        sel_s[...] = jnp.where(tm, m, sel_s[...])
        sel_x1[...] = jnp.where(tm, bx1, sel_x1[...])
        sel_y1[...] = jnp.where(tm, by1, sel_y1[...])
        sel_x2[...] = jnp.where(tm, bx2, sel_x2[...])
        sel_y2[...] = jnp.where(tm, by2, sel_y2[...])
        return carry

    lax.fori_loop(0, MAXDET, nms_step, 0)

    row = lax.broadcasted_iota(i32, (R, PD), 0)
    pos = row * PD + lane_pd
    lane8 = lax.broadcasted_iota(i32, (1, 8), 1)
    lane4 = lax.broadcasted_iota(i32, (1, 4), 1)
    brow = [row < C, row >= C]

    def _reduce2(x, fn):
        return fn(fn(x, axis=1, keepdims=True), axis=0, keepdims=True)

    def ext_step(k, carry):
        flat = sel_s[...]
        ohs = []
        scs = []
        lbs = []
        vbs = []
        for b in range(B):
            sb = jnp.where(brow[b], flat, NEG)
            m_b = _reduce2(sb, jnp.max)
            eqb = sb == m_b
            pmin = _reduce2(jnp.where(eqb, pos, R * PD), jnp.min)
            oh = pos == pmin
            ohs.append(oh)
            label = (pmin // PD) - b * C
            gx1 = _reduce2(jnp.where(oh, sel_x1[...], 0.0), jnp.sum)
            gy1 = _reduce2(jnp.where(oh, sel_y1[...], 0.0), jnp.sum)
            gx2 = _reduce2(jnp.where(oh, sel_x2[...], 0.0), jnp.sum)
            gy2 = _reduce2(jnp.where(oh, sel_y2[...], 0.0), jnp.sum)
            vb = m_b > (NEG * 0.5)
            scs.append(jnp.where(vb, m_b, -1.0))
            lbs.append(jnp.where(vb, label, -1))
            vbs.append(vb)
            bvec = jnp.where(lane4 == 0, gx1,
                   jnp.where(lane4 == 1, gy1,
                   jnp.where(lane4 == 2, gx2, gy2)))
            bvec = jnp.where(vb, bvec, -1.0)
            boxes_out[b, pl.ds(k, 1), :] = bvec
        sel_s[...] = jnp.where(ohs[0] | ohs[1], NEG, flat)
        sc_row = jnp.where(lane8 == 0, scs[0],
                 jnp.where(lane8 == 1, scs[1], -1.0))
        lb_row = jnp.where(lane8 == 0, lbs[0],
                 jnp.where(lane8 == 1, lbs[1], -1))
        scoresT_out[pl.ds(k, 1), :] = sc_row
        labelsT_out[pl.ds(k, 1), :] = lb_row
        return carry

    lax.fori_loop(0, MAXDET, ext_step, 0)


@jax.jit
def kernel(boxes, classification):
    f32 = jnp.float32
    scores = jnp.transpose(classification, (0, 2, 1)).reshape(R, N)
    scores = jnp.pad(scores, ((0, 0), (0, NP - N)), constant_values=-1.0)
    coords = []
    for c in range(4):
        xc = boxes[:, :, c]
        xc = jnp.broadcast_to(xc[:, None, :], (B, C, N)).reshape(R, N)
        coords.append(jnp.pad(xc, ((0, 0), (0, NP - N))))
    out_shapes = (
        jax.ShapeDtypeStruct((B, PD, 4), f32),
        jax.ShapeDtypeStruct((PD, 8), f32),
        jax.ShapeDtypeStruct((PD, 8), jnp.int32),
    )
    scratch = [
        pltpu.VMEM((R, NP), f32),
        pltpu.VMEM((R, NP), f32),
        pltpu.VMEM((R, PD), f32),
        pltpu.VMEM((R, PD), f32),
        pltpu.VMEM((R, PD), f32),
        pltpu.VMEM((R, PD), f32),
        pltpu.VMEM((R, PD), f32),
    ]
    bxs, scT, lbT = pl.pallas_call(
        _fd_kernel,
        out_shape=out_shapes,
        scratch_shapes=scratch,
    )(scores, *coords)
    filtered_boxes = bxs[:, :MAXDET, :]
    filtered_scores = jnp.transpose(scT[:MAXDET, :B], (1, 0))
    filtered_labels = jnp.transpose(lbT[:MAXDET, :B], (1, 0)).astype(jnp.int32)
    return filtered_boxes, filtered_scores, filtered_labels


# speedup vs baseline: 15.1192x; 1.0934x over previous
import jax
import jax.numpy as jnp
from jax import lax
from jax.experimental import pallas as pl
from jax.experimental.pallas import tpu as pltpu

NMS_THRESHOLD = 0.5
SCORE_THRESHOLD = 0.05
MAXDET = 300
NEG = -1e9
B, N, C = 2, 5000, 20
NP = 5120
PD = 304
R = B * C


def _fd_kernel(s_in, x1_in, y1_in, x2_in, y2_in,
               boxes_out, scoresT_out, labelsT_out,
               s_ref, area_ref, sel_s,
               sel_x1, sel_y1, sel_x2, sel_y2):
    f32 = jnp.float32
    i32 = jnp.int32
    iota_n = lax.broadcasted_iota(i32, (R, NP), 1)
    lane_pd = lax.broadcasted_iota(i32, (R, PD), 1)

    x1 = x1_in[...]
    y1 = y1_in[...]
    x2 = x2_in[...]
    y2 = y2_in[...]
    raw = s_in[...]
    s_ref[...] = jnp.where(raw > SCORE_THRESHOLD, raw, NEG)
    area_ref[...] = (x2 - x1) * (y2 - y1)
    sel_s[...] = jnp.full((R, PD), NEG, f32)
    sel_x1[...] = jnp.zeros((R, PD), f32)
    sel_y1[...] = jnp.zeros((R, PD), f32)
    sel_x2[...] = jnp.zeros((R, PD), f32)
    sel_y2[...] = jnp.zeros((R, PD), f32)
    area = area_ref[...]

    def nms_step(t, carry):
        s = s_ref[...]
        m = jnp.max(s, axis=1, keepdims=True)
        eq = s == m
        idx = jnp.min(jnp.where(eq, iota_n, NP), axis=1, keepdims=True)
        onehot = iota_n == idx
        bx1 = jnp.sum(jnp.where(onehot, x1, 0.0), axis=1, keepdims=True)
        by1 = jnp.sum(jnp.where(onehot, y1, 0.0), axis=1, keepdims=True)
        bx2 = jnp.sum(jnp.where(onehot, x2, 0.0), axis=1, keepdims=True)
        by2 = jnp.sum(jnp.where(onehot, y2, 0.0), axis=1, keepdims=True)
        valid = m > (NEG * 0.5)
        xx1 = jnp.maximum(bx1, x1)
        yy1 = jnp.maximum(by1, y1)
        xx2 = jnp.minimum(bx2, x2)
        yy2 = jnp.minimum(by2, y2)
        inter = jnp.maximum(xx2 - xx1, 0.0) * jnp.maximum(yy2 - yy1, 0.0)
        area_a = (bx2 - bx1) * (by2 - by1)
        union = area_a + area - inter
        supp = (inter / union) > NMS_THRESHOLD
        s_ref[...] = jnp.where((valid & supp) | onehot, NEG, s)
        tm = lane_pd == t
        sel_s[...] = jnp.where(tm, m, sel_s[...])
        sel_x1[...] = jnp.where(tm, bx1, sel_x1[...])
        sel_y1[...] = jnp.where(tm, by1, sel_y1[...])
        sel_x2[...] = jnp.where(tm, bx2, sel_x2[...])
        sel_y2[...] = jnp.where(tm, by2, sel_y2[...])
        return carry

    lax.fori_loop(0, MAXDET, nms_step, 0)

    row = lax.broadcasted_iota(i32, (R, PD), 0)
    pos = row * PD + lane_pd
    lane8 = lax.broadcasted_iota(i32, (1, 8), 1)
    lane4 = lax.broadcasted_iota(i32, (1, 4), 1)
    brow = [row < C, row >= C]

    def _reduce2(x, fn):
        return fn(fn(x, axis=1, keepdims=True), axis=0, keepdims=True)

    cx1 = sel_x1[...]
    cy1 = sel_y1[...]
    cx2 = sel_x2[...]
    cy2 = sel_y2[...]

    def ext_step(k, carry):
        flat = sel_s[...]
        ohs = []
        scs = []
        lbs = []
        vbs = []
        for b in range(B):
            sb = jnp.where(brow[b], flat, NEG)
            m_b = _reduce2(sb, jnp.max)
            eqb = sb == m_b
            pmin = _reduce2(jnp.where(eqb, pos, R * PD), jnp.min)
            oh = pos == pmin
            ohs.append(oh)
            label = (pmin // PD) - b * C
            gx1 = _reduce2(jnp.where(oh, cx1, 0.0), jnp.sum)
            gy1 = _reduce2(jnp.where(oh, cy1, 0.0), jnp.sum)
            gx2 = _reduce2(jnp.where(oh, cx2, 0.0), jnp.sum)
            gy2 = _reduce2(jnp.where(oh, cy2, 0.0), jnp.sum)
            vb = m_b > (NEG * 0.5)
            scs.append(jnp.where(vb, m_b, -1.0))
            lbs.append(jnp.where(vb, label, -1))
            vbs.append(vb)
            bvec = jnp.where(lane4 == 0, gx1,
                   jnp.where(lane4 == 1, gy1,
                   jnp.where(lane4 == 2, gx2, gy2)))
            bvec = jnp.where(vb, bvec, -1.0)
            boxes_out[b, pl.ds(k, 1), :] = bvec
        sel_s[...] = jnp.where(ohs[0] | ohs[1], NEG, flat)
        sc_row = jnp.where(lane8 == 0, scs[0],
                 jnp.where(lane8 == 1, scs[1], -1.0))
        lb_row = jnp.where(lane8 == 0, lbs[0],
                 jnp.where(lane8 == 1, lbs[1], -1))
        scoresT_out[pl.ds(k, 1), :] = sc_row
        labelsT_out[pl.ds(k, 1), :] = lb_row
        return carry

    lax.fori_loop(0, MAXDET, ext_step, 0)


@jax.jit
def kernel(boxes, classification):
    f32 = jnp.float32
    scores = jnp.transpose(classification, (0, 2, 1)).reshape(R, N)
    scores = jnp.pad(scores, ((0, 0), (0, NP - N)), constant_values=-1.0)
    coords = []
    for c in range(4):
        xc = boxes[:, :, c]
        xc = jnp.broadcast_to(xc[:, None, :], (B, C, N)).reshape(R, N)
        coords.append(jnp.pad(xc, ((0, 0), (0, NP - N))))
    out_shapes = (
        jax.ShapeDtypeStruct((B, PD, 4), f32),
        jax.ShapeDtypeStruct((PD, 8), f32),
        jax.ShapeDtypeStruct((PD, 8), jnp.int32),
    )
    scratch = [
        pltpu.VMEM((R, NP), f32),
        pltpu.VMEM((R, NP), f32),
        pltpu.VMEM((R, PD), f32),
        pltpu.VMEM((R, PD), f32),
        pltpu.VMEM((R, PD), f32),
        pltpu.VMEM((R, PD), f32),
        pltpu.VMEM((R, PD), f32),
    ]
    bxs, scT, lbT = pl.pallas_call(
        _fd_kernel,
        out_shape=out_shapes,
        scratch_shapes=scratch,
    )(scores, *coords)
    filtered_boxes = bxs[:, :MAXDET, :]
    filtered_scores = jnp.transpose(scT[:MAXDET, :B], (1, 0))
    filtered_labels = jnp.transpose(lbT[:MAXDET, :B], (1, 0)).astype(jnp.int32)
    return filtered_boxes, filtered_scores, filtered_labels
